# baseline (speedup 1.0000x reference)
import jax
import jax.numpy as jnp
from jax.experimental import pallas as pl
from jax.experimental.pallas import tpu as pltpu

BK = 3584
HEAD = 128


def _mlp_kernel(x_ref, w1_ref, b1_ref, w2_ref, b2_ref, wh_ref, bh_ref,
                out_ref, acc_ref):
    k = pl.program_id(0)
    nk = pl.num_programs(0)

    nchunk = 512

    @pl.when(k == 0)
    def _():
        for c in range(0, w1_ref.shape[1], nchunk):
            acc_ref[:, c:c + nchunk] = jnp.dot(
                x_ref[...], w1_ref[:, c:c + nchunk],
                preferred_element_type=jnp.float32)

    @pl.when(k > 0)
    def _():
        for c in range(0, w1_ref.shape[1], nchunk):
            acc_ref[:, c:c + nchunk] += jnp.dot(
                x_ref[...], w1_ref[:, c:c + nchunk],
                preferred_element_type=jnp.float32)

    @pl.when(k == nk - 1)
    def _():
        rows = acc_ref.shape[0]
        chunk = 200
        for c in range(rows // chunk):
            sl = slice(c * chunk, (c + 1) * chunk)
            h1 = jnp.maximum(acc_ref[sl, :] + b1_ref[...], 0.0)
            h2 = jnp.maximum(
                jnp.dot(h1.astype(jnp.bfloat16), w2_ref[...],
                        preferred_element_type=jnp.float32)
                + b2_ref[...], 0.0)
            out_ref[sl, :] = (jnp.dot(h2.astype(jnp.bfloat16), wh_ref[...],
                                      preferred_element_type=jnp.float32)
                              + bh_ref[...])


def kernel(feature_vectors, W1, b1, W2, b2, Wc, bc, Wr, br):
    n, d_in = feature_vectors.shape
    hid = W1.shape[1]
    nc = Wc.shape[1]
    nr = Wr.shape[1]

    wh = jnp.pad(jnp.concatenate([Wc, Wr], axis=1),
                 ((0, 0), (0, HEAD - nc - nr))).astype(jnp.bfloat16)
    bh = jnp.pad(jnp.concatenate([bc, br]), (0, HEAD - nc - nr)).reshape(1, HEAD)
    b1r = b1.reshape(1, hid)
    b2r = b2.reshape(1, hid)
    w2c = W2.astype(jnp.bfloat16)

    grid = (d_in // BK,)
    out = pl.pallas_call(
        _mlp_kernel,
        grid=grid,
        in_specs=[
            pl.BlockSpec((n, BK), lambda k: (0, k)),
            pl.BlockSpec((BK, hid), lambda k: (k, 0)),
            pl.BlockSpec((1, hid), lambda k: (0, 0)),
            pl.BlockSpec((hid, hid), lambda k: (0, 0)),
            pl.BlockSpec((1, hid), lambda k: (0, 0)),
            pl.BlockSpec((hid, HEAD), lambda k: (0, 0)),
            pl.BlockSpec((1, HEAD), lambda k: (0, 0)),
        ],
        out_specs=pl.BlockSpec((n, HEAD), lambda k: (0, 0)),
        out_shape=jax.ShapeDtypeStruct((n, HEAD), jnp.float32),
        scratch_shapes=[
            pltpu.VMEM((n, hid), jnp.float32),
        ],
        compiler_params=pltpu.CompilerParams(
            dimension_semantics=("arbitrary",),
            vmem_limit_bytes=67_000_000,
        ),
    )(feature_vectors, W1, b1r, w2c, b2r, wh, bh)
    return out[:, :nc], out[:, nc:nc + nr]


# speedup vs baseline: 1.0027x; 1.0027x over previous
import jax
import jax.numpy as jnp
from jax.experimental import pallas as pl
from jax.experimental.pallas import tpu as pltpu

BK = 1792
NK = 28
DEPTH = 2
HEAD = 128
NCHUNK = 256


def _mlp_kernel(x_hbm, w1_hbm, b1_ref, w2_ref, b2_ref, wh_ref, bh_ref,
                out_ref, acc_ref, xb0, xb1, wb0, wb1, xsem, wsem):
    xbufs = (xb0, xb1)
    wbufs = (wb0, wb1)

    def x_copy(i, s):
        return pltpu.make_async_copy(
            x_hbm.at[:, pl.ds(i * BK, BK)], xbufs[s], xsem.at[s])

    def w_copy(i, s):
        return pltpu.make_async_copy(
            w1_hbm.at[pl.ds(i * BK, BK), :], wbufs[s], wsem.at[s])

    acc_ref[...] = jnp.zeros_like(acc_ref)
    for s in range(DEPTH):
        x_copy(s, s).start()
        w_copy(s, s).start()

    ngroups = NK // DEPTH

    def group(g, carry):
        for s in range(DEPTH):
            i = g * DEPTH + s
            x_copy(i, s).wait()
            w_copy(i, s).wait()
            for c in range(0, acc_ref.shape[1], NCHUNK):
                acc_ref[:, c:c + NCHUNK] += jnp.dot(
                    xbufs[s][...], wbufs[s][:, c:c + NCHUNK],
                    preferred_element_type=jnp.float32)

            @pl.when(g < ngroups - 1)
            def _():
                x_copy(i + DEPTH, s).start()
                w_copy(i + DEPTH, s).start()
        return carry

    jax.lax.fori_loop(0, ngroups, group, 0)

    rows = acc_ref.shape[0]
    chunk = 200
    for r in range(rows // chunk):
        sl = slice(r * chunk, (r + 1) * chunk)
        h1 = jnp.maximum(acc_ref[sl, :] + b1_ref[...], 0.0)
        h2 = jnp.maximum(
            jnp.dot(h1.astype(jnp.bfloat16), w2_ref[...],
                    preferred_element_type=jnp.float32)
            + b2_ref[...], 0.0)
        out_ref[sl, :] = (jnp.dot(h2.astype(jnp.bfloat16), wh_ref[...],
                                  preferred_element_type=jnp.float32)
                          + bh_ref[...])


def kernel(feature_vectors, W1, b1, W2, b2, Wc, bc, Wr, br):
    n, d_in = feature_vectors.shape
    hid = W1.shape[1]
    nc = Wc.shape[1]
    nr = Wr.shape[1]

    wh = jnp.pad(jnp.concatenate([Wc, Wr], axis=1),
                 ((0, 0), (0, HEAD - nc - nr))).astype(jnp.bfloat16)
    bh = jnp.pad(jnp.concatenate([bc, br]), (0, HEAD - nc - nr)).reshape(1, HEAD)
    b1r = b1.reshape(1, hid)
    b2r = b2.reshape(1, hid)
    w2c = W2.astype(jnp.bfloat16)

    out = pl.pallas_call(
        _mlp_kernel,
        in_specs=[
            pl.BlockSpec(memory_space=pl.ANY),
            pl.BlockSpec(memory_space=pl.ANY),
            pl.BlockSpec(memory_space=pltpu.MemorySpace.VMEM),
            pl.BlockSpec(memory_space=pltpu.MemorySpace.VMEM),
            pl.BlockSpec(memory_space=pltpu.MemorySpace.VMEM),
            pl.BlockSpec(memory_space=pltpu.MemorySpace.VMEM),
            pl.BlockSpec(memory_space=pltpu.MemorySpace.VMEM),
        ],
        out_specs=pl.BlockSpec(memory_space=pltpu.MemorySpace.VMEM),
        out_shape=jax.ShapeDtypeStruct((n, HEAD), jnp.float32),
        scratch_shapes=[
            pltpu.VMEM((n, hid), jnp.float32),
            pltpu.VMEM((n, BK), jnp.float32),
            pltpu.VMEM((n, BK), jnp.float32),
            pltpu.VMEM((BK, hid), jnp.float32),
            pltpu.VMEM((BK, hid), jnp.float32),
            pltpu.SemaphoreType.DMA((DEPTH,)),
            pltpu.SemaphoreType.DMA((DEPTH,)),
        ],
        compiler_params=pltpu.CompilerParams(
            vmem_limit_bytes=67_000_000,
        ),
    )(feature_vectors, W1, b1r, w2c, b2r, wh, bh)
    return out[:, :nc], out[:, nc:nc + nr]


# speedup vs baseline: 1.0097x; 1.0070x over previous
import jax
import jax.numpy as jnp
from jax.experimental import pallas as pl
from jax.experimental.pallas import tpu as pltpu

BK = 3584
NK = 14
DEPTH = 2
HEAD = 128
NCHUNK = 256


def _mlp_kernel(x_hbm, w1_hbm, b1_ref, w2_ref, b2_ref, wh_ref, bh_ref,
                out_ref, acc_ref, xb0, xb1, wb0, wb1, xsem, wsem):
    xbufs = (xb0, xb1)
    wbufs = (wb0, wb1)

    def x_copy(i, s):
        return pltpu.make_async_copy(
            x_hbm.at[:, pl.ds(i * BK, BK)], xbufs[s], xsem.at[s])

    def w_copy(i, s):
        return pltpu.make_async_copy(
            w1_hbm.at[pl.ds(i * BK, BK), :], wbufs[s], wsem.at[s])

    acc_ref[...] = jnp.zeros_like(acc_ref)
    for s in range(DEPTH):
        x_copy(s, s).start()
        w_copy(s, s).start()

    ngroups = NK // DEPTH

    def group(g, carry):
        for s in range(DEPTH):
            i = g * DEPTH + s
            x_copy(i, s).wait()
            w_copy(i, s).wait()
            for c in range(0, acc_ref.shape[1], NCHUNK):
                acc_ref[:, c:c + NCHUNK] += jnp.dot(
                    xbufs[s][...], wbufs[s][:, c:c + NCHUNK],
                    preferred_element_type=jnp.float32)

            @pl.when(g < ngroups - 1)
            def _():
                x_copy(i + DEPTH, s).start()
                w_copy(i + DEPTH, s).start()
        return carry

    jax.lax.fori_loop(0, ngroups, group, 0)

    rows = acc_ref.shape[0]
    chunk = 200
    for r in range(rows // chunk):
        sl = slice(r * chunk, (r + 1) * chunk)
        h1 = jnp.maximum(acc_ref[sl, :] + b1_ref[...], 0.0)
        h2 = jnp.maximum(
            jnp.dot(h1.astype(jnp.bfloat16), w2_ref[...],
                    preferred_element_type=jnp.float32)
            + b2_ref[...], 0.0)
        out_ref[sl, :] = (jnp.dot(h2.astype(jnp.bfloat16), wh_ref[...],
                                  preferred_element_type=jnp.float32)
                          + bh_ref[...])


def kernel(feature_vectors, W1, b1, W2, b2, Wc, bc, Wr, br):
    n, d_in = feature_vectors.shape
    hid = W1.shape[1]
    nc = Wc.shape[1]
    nr = Wr.shape[1]

    wh = jnp.pad(jnp.concatenate([Wc, Wr], axis=1),
                 ((0, 0), (0, HEAD - nc - nr))).astype(jnp.bfloat16)
    bh = jnp.pad(jnp.concatenate([bc, br]), (0, HEAD - nc - nr)).reshape(1, HEAD)
    b1r = b1.reshape(1, hid)
    b2r = b2.reshape(1, hid)
    w2c = W2.astype(jnp.bfloat16)

    out = pl.pallas_call(
        _mlp_kernel,
        in_specs=[
            pl.BlockSpec(memory_space=pl.ANY),
            pl.BlockSpec(memory_space=pl.ANY),
            pl.BlockSpec(memory_space=pltpu.MemorySpace.VMEM),
            pl.BlockSpec(memory_space=pltpu.MemorySpace.VMEM),
            pl.BlockSpec(memory_space=pltpu.MemorySpace.VMEM),
            pl.BlockSpec(memory_space=pltpu.MemorySpace.VMEM),
            pl.BlockSpec(memory_space=pltpu.MemorySpace.VMEM),
        ],
        out_specs=pl.BlockSpec(memory_space=pltpu.MemorySpace.VMEM),
        out_shape=jax.ShapeDtypeStruct((n, HEAD), jnp.float32),
        scratch_shapes=[
            pltpu.VMEM((n, hid), jnp.float32),
            pltpu.VMEM((n, BK), jnp.float32),
            pltpu.VMEM((n, BK), jnp.float32),
            pltpu.VMEM((BK, hid), jnp.float32),
            pltpu.VMEM((BK, hid), jnp.float32),
            pltpu.SemaphoreType.DMA((DEPTH,)),
            pltpu.SemaphoreType.DMA((DEPTH,)),
        ],
        compiler_params=pltpu.CompilerParams(
            vmem_limit_bytes=67_000_000,
        ),
    )(feature_vectors, W1, b1r, w2c, b2r, wh, bh)
    return out[:, :nc], out[:, nc:nc + nr]
